# baseline (speedup 1.0000x reference)
import jax
import jax.numpy as jnp
from jax.experimental import pallas as pl
from jax.experimental.pallas import tpu as pltpu

_PATCH = [
    (0, 1, 0.5),
    (3, 2, 1.5),
    (10, 20, 0.25),
    (100, 200, 2.0),
    (7, 7, 0.1),
    (500, 1000, 0.9),
    (2048, 4095, 1.2),
    (4095, 0, 0.3),
]

_N = 4096
_BLOCK_ROWS = 256


def _body(x_ref, o_ref):
    row0 = pl.program_id(0) * _BLOCK_ROWS
    x = x_ref[...]
    y = x + 1.0
    rows = jax.lax.broadcasted_iota(jnp.int32, (_BLOCK_ROWS, _N), 0) + row0
    cols = jax.lax.broadcasted_iota(jnp.int32, (_BLOCK_ROWS, _N), 1)
    for i, j, v in _PATCH:
        y = jnp.where((rows == j) & (cols == i), x + v, y)
    o_ref[...] = y


def kernel(X):
    x2d = X.reshape(_N, _N)
    y2d = pl.pallas_call(
        _body,
        grid=(_N // _BLOCK_ROWS,),
        in_specs=[pl.BlockSpec((_BLOCK_ROWS, _N), lambda r: (r, 0))],
        out_specs=pl.BlockSpec((_BLOCK_ROWS, _N), lambda r: (r, 0)),
        out_shape=jax.ShapeDtypeStruct((_N, _N), X.dtype),
        compiler_params=pltpu.CompilerParams(
            dimension_semantics=("arbitrary",),
        ),
    )(x2d)
    return y2d.reshape(X.shape)


# speedup vs baseline: 1.0031x; 1.0031x over previous
import jax
import jax.numpy as jnp
from jax.experimental import pallas as pl
from jax.experimental.pallas import tpu as pltpu

_PATCH = [
    (0, 1, 0.5),
    (3, 2, 1.5),
    (10, 20, 0.25),
    (100, 200, 2.0),
    (7, 7, 0.1),
    (500, 1000, 0.9),
    (2048, 4095, 1.2),
    (4095, 0, 0.3),
]

_N = 4096
_BLOCK_ROWS = 256


def _body(x_ref, o_ref):
    pid = pl.program_id(0)
    o_ref[...] = x_ref[...] + 1.0
    cols = jax.lax.broadcasted_iota(jnp.int32, (1, _N), 1)
    for i, j, v in _PATCH:
        blk, jl = divmod(j, _BLOCK_ROWS)

        @pl.when(pid == blk)
        def _(jl=jl, i=i, v=v):
            row = x_ref[jl : jl + 1, :]
            o_ref[jl : jl + 1, :] = jnp.where(cols == i, row + v, row + 1.0)


def kernel(X):
    x2d = X.reshape(_N, _N)
    y2d = pl.pallas_call(
        _body,
        grid=(_N // _BLOCK_ROWS,),
        in_specs=[pl.BlockSpec((_BLOCK_ROWS, _N), lambda r: (r, 0))],
        out_specs=pl.BlockSpec((_BLOCK_ROWS, _N), lambda r: (r, 0)),
        out_shape=jax.ShapeDtypeStruct((_N, _N), X.dtype),
        compiler_params=pltpu.CompilerParams(
            dimension_semantics=("arbitrary",),
        ),
    )(x2d)
    return y2d.reshape(X.shape)


# speedup vs baseline: 1.0109x; 1.0077x over previous
import jax
import jax.numpy as jnp
from jax.experimental import pallas as pl
from jax.experimental.pallas import tpu as pltpu

_PATCH = [
    (0, 1, 0.5),
    (3, 2, 1.5),
    (10, 20, 0.25),
    (100, 200, 2.0),
    (7, 7, 0.1),
    (500, 1000, 0.9),
    (2048, 4095, 1.2),
    (4095, 0, 0.3),
]

_N = 4096
_BLOCK_ROWS = 512


def _body(x_ref, o_ref):
    pid = pl.program_id(0)
    o_ref[...] = x_ref[...] + 1.0
    cols = jax.lax.broadcasted_iota(jnp.int32, (1, _N), 1)
    for i, j, v in _PATCH:
        blk, jl = divmod(j, _BLOCK_ROWS)

        @pl.when(pid == blk)
        def _(jl=jl, i=i, v=v):
            row = x_ref[jl : jl + 1, :]
            o_ref[jl : jl + 1, :] = jnp.where(cols == i, row + v, row + 1.0)


def kernel(X):
    x2d = X.reshape(_N, _N)
    y2d = pl.pallas_call(
        _body,
        grid=(_N // _BLOCK_ROWS,),
        in_specs=[pl.BlockSpec((_BLOCK_ROWS, _N), lambda r: (r, 0))],
        out_specs=pl.BlockSpec((_BLOCK_ROWS, _N), lambda r: (r, 0)),
        out_shape=jax.ShapeDtypeStruct((_N, _N), X.dtype),
        compiler_params=pltpu.CompilerParams(
            dimension_semantics=("parallel",),
        ),
    )(x2d)
    return y2d.reshape(X.shape)
